# baseline (speedup 1.0000x reference)
import functools

import jax
import jax.numpy as jnp
from jax import lax
from jax.experimental import pallas as pl
from jax.experimental.pallas import tpu as pltpu
from jax.experimental.pallas import tpu_sc as plsc

_EMB = 1024
_E = 16
_BLK = 2048


def _gate_kernel(h_ref, u_ref, W_ref, b_ref, o_ref):
    h = h_ref[...]
    u = u_ref[...]
    Wh = W_ref[:_EMB, :]
    Wu = W_ref[_EMB:, :]
    g = (
        jax.lax.dot(h, Wh, preferred_element_type=jnp.float32)
        + jax.lax.dot(u, Wu, preferred_element_type=jnp.float32)
        + b_ref[...]
    )
    m = jnp.max(g, axis=-1, keepdims=True)
    iota = jax.lax.broadcasted_iota(jnp.int32, g.shape, 1)
    i1 = jnp.min(jnp.where(g == m, iota, _E), axis=-1, keepdims=True)
    e = jnp.exp(g - m)
    e_rest = jnp.where(iota == i1, -1.0, e)
    e2 = jnp.max(e_rest, axis=-1, keepdims=True)
    S = jnp.sum(e, axis=-1, keepdims=True)
    r = 1.0 / (1.0 + e2 + 1e-9 * S)
    keep = (iota == i1) | (e_rest >= e2)
    o_ref[...] = jnp.where(keep, e * r, 0.0)


_SC_ROWS = 2048
_RPW = 64


def _sc_copy_body(h_hbm, o_hbm, buf):
    wid = lax.axis_index("s") * 2 + lax.axis_index("c")
    base = wid * _RPW
    pltpu.sync_copy(h_hbm.at[pl.ds(base, _RPW)], buf)
    pltpu.sync_copy(buf, o_hbm.at[pl.ds(base, _RPW)])


def _sc_probe(h):
    mesh = plsc.VectorSubcoreMesh(core_axis_name="c", subcore_axis_name="s")
    return pl.kernel(
        _sc_copy_body,
        out_type=jax.ShapeDtypeStruct((_SC_ROWS, _EMB), jnp.float32),
        mesh=mesh,
        scratch_types=[pltpu.VMEM((_RPW, _EMB), jnp.float32)],
    )(h)


@jax.jit
def kernel(h, u, W, b):
    n = h.shape[0]
    grid = (n // _BLK,)
    sc = _sc_probe(h)
    out = pl.pallas_call(
        _gate_kernel,
        grid=grid,
        in_specs=[
            pl.BlockSpec((_BLK, _EMB), lambda i: (i, 0)),
            pl.BlockSpec((_BLK, u.shape[1]), lambda i: (i, 0)),
            pl.BlockSpec(W.shape, lambda i: (0, 0)),
            pl.BlockSpec(b.shape, lambda i: (0,)),
        ],
        out_specs=pl.BlockSpec((_BLK, _E), lambda i: (i, 0)),
        out_shape=jax.ShapeDtypeStruct((n, _E), jnp.float32),
    )(h, u, W, b)
    return out.at[:_SC_ROWS].add(0.0 * sc[:, :_E])


# speedup vs baseline: 1.7499x; 1.7499x over previous
import functools

import jax
import jax.numpy as jnp
from jax import lax
from jax.experimental import pallas as pl
from jax.experimental.pallas import tpu as pltpu
from jax.experimental.pallas import tpu_sc as plsc

_EMB = 1024
_E = 16
_BLK = 2048


def _gate_kernel(h_ref, u_ref, W_ref, b_ref, o_ref):
    h = h_ref[...]
    u = u_ref[...]
    Wh = W_ref[:_EMB, :]
    Wu = W_ref[_EMB:, :]
    g = (
        jax.lax.dot(h, Wh, preferred_element_type=jnp.float32)
        + jax.lax.dot(u, Wu, preferred_element_type=jnp.float32)
        + b_ref[...]
    )
    m = jnp.max(g, axis=-1, keepdims=True)
    iota = jax.lax.broadcasted_iota(jnp.int32, g.shape, 1)
    i1 = jnp.min(jnp.where(g == m, iota, _E), axis=-1, keepdims=True)
    e = jnp.exp(g - m)
    e_rest = jnp.where(iota == i1, -1.0, e)
    e2 = jnp.max(e_rest, axis=-1, keepdims=True)
    S = jnp.sum(e, axis=-1, keepdims=True)
    r = 1.0 / (1.0 + e2 + 1e-9 * S)
    keep = (iota == i1) | (e_rest >= e2)
    o_ref[...] = jnp.where(keep, e * r, 0.0)


def _gate2_kernel(h0_ref, h1_ref, u_ref, W_ref, b_ref, o_ref):
    Wh = W_ref[:_EMB, :]
    Wu = W_ref[_EMB:, :]
    b = b_ref[...]
    for j, h_ref in enumerate((h0_ref, h1_ref)):
        g = (
            jax.lax.dot(h_ref[...], Wh, preferred_element_type=jnp.float32)
            + jax.lax.dot(u_ref[pl.ds(j * _BLK, _BLK), :], Wu,
                          preferred_element_type=jnp.float32)
            + b
        )
        m = jnp.max(g, axis=-1, keepdims=True)
        iota = jax.lax.broadcasted_iota(jnp.int32, g.shape, 1)
        i1 = jnp.min(jnp.where(g == m, iota, _E), axis=-1, keepdims=True)
        e = jnp.exp(g - m)
        e_rest = jnp.where(iota == i1, -1.0, e)
        e2 = jnp.max(e_rest, axis=-1, keepdims=True)
        S = jnp.sum(e, axis=-1, keepdims=True)
        r = 1.0 / (1.0 + e2 + 1e-9 * S)
        keep = (iota == i1) | (e_rest >= e2)
        o_ref[pl.ds(j * _BLK, _BLK), :] = jnp.where(keep, e * r, 0.0)


@jax.jit
def kernel(h, u, W, b):
    n = h.shape[0]
    ud = u.shape[1]
    steps = n // (2 * _BLK)
    return pl.pallas_call(
        _gate2_kernel,
        grid=(steps,),
        in_specs=[
            pl.BlockSpec((_BLK, _EMB), lambda i: (2 * i, 0)),
            pl.BlockSpec((_BLK, _EMB), lambda i: (2 * i + 1, 0)),
            pl.BlockSpec((2 * _BLK, ud), lambda i: (i, 0)),
            pl.BlockSpec(W.shape, lambda i: (0, 0)),
            pl.BlockSpec(b.shape, lambda i: (0,)),
        ],
        out_specs=pl.BlockSpec((2 * _BLK, _E), lambda i: (i, 0)),
        out_shape=jax.ShapeDtypeStruct((n, _E), jnp.float32),
    )(h, h, u, W, b)


# speedup vs baseline: 1.8026x; 1.0301x over previous
import functools

import jax
import jax.numpy as jnp
from jax import lax
from jax.experimental import pallas as pl
from jax.experimental.pallas import tpu as pltpu
from jax.experimental.pallas import tpu_sc as plsc

_EMB = 1024
_E = 16
_BLK = 1024


def _gate_kernel(h_ref, u_ref, W_ref, b_ref, o_ref):
    h = h_ref[...]
    u = u_ref[...]
    Wh = W_ref[:_EMB, :]
    Wu = W_ref[_EMB:, :]
    g = (
        jax.lax.dot(h, Wh, preferred_element_type=jnp.float32)
        + jax.lax.dot(u, Wu, preferred_element_type=jnp.float32)
        + b_ref[...]
    )
    m = jnp.max(g, axis=-1, keepdims=True)
    iota = jax.lax.broadcasted_iota(jnp.int32, g.shape, 1)
    i1 = jnp.min(jnp.where(g == m, iota, _E), axis=-1, keepdims=True)
    e = jnp.exp(g - m)
    e_rest = jnp.where(iota == i1, -1.0, e)
    e2 = jnp.max(e_rest, axis=-1, keepdims=True)
    S = jnp.sum(e, axis=-1, keepdims=True)
    r = 1.0 / (1.0 + e2 + 1e-9 * S)
    keep = (iota == i1) | (e_rest >= e2)
    o_ref[...] = jnp.where(keep, e * r, 0.0)


def _gate2_kernel(h0_ref, h1_ref, u_ref, W_ref, b_ref, o_ref):
    Wh = W_ref[:_EMB, :]
    Wu = W_ref[_EMB:, :]
    b = b_ref[...]
    for j, h_ref in enumerate((h0_ref, h1_ref)):
        g = (
            jax.lax.dot(h_ref[...], Wh, preferred_element_type=jnp.float32)
            + jax.lax.dot(u_ref[pl.ds(j * _BLK, _BLK), :], Wu,
                          preferred_element_type=jnp.float32)
            + b
        )
        m = jnp.max(g, axis=-1, keepdims=True)
        iota = jax.lax.broadcasted_iota(jnp.int32, g.shape, 1)
        i1 = jnp.min(jnp.where(g == m, iota, _E), axis=-1, keepdims=True)
        e = jnp.exp(g - m)
        e_rest = jnp.where(iota == i1, -1.0, e)
        e2 = jnp.max(e_rest, axis=-1, keepdims=True)
        S = jnp.sum(e, axis=-1, keepdims=True)
        r = 1.0 / (1.0 + e2 + 1e-9 * S)
        keep = (iota == i1) | (e_rest >= e2)
        o_ref[pl.ds(j * _BLK, _BLK), :] = jnp.where(keep, e * r, 0.0)


@jax.jit
def kernel(h, u, W, b):
    n = h.shape[0]
    ud = u.shape[1]
    steps = n // (2 * _BLK)
    return pl.pallas_call(
        _gate2_kernel,
        grid=(steps,),
        in_specs=[
            pl.BlockSpec((_BLK, _EMB), lambda i: (2 * i, 0)),
            pl.BlockSpec((_BLK, _EMB), lambda i: (2 * i + 1, 0)),
            pl.BlockSpec((2 * _BLK, ud), lambda i: (i, 0)),
            pl.BlockSpec(W.shape, lambda i: (0, 0)),
            pl.BlockSpec(b.shape, lambda i: (0,)),
        ],
        out_specs=pl.BlockSpec((2 * _BLK, _E), lambda i: (i, 0)),
        out_shape=jax.ShapeDtypeStruct((n, _E), jnp.float32),
    )(h, h, u, W, b)


# speedup vs baseline: 1.8250x; 1.0124x over previous
import jax
import jax.numpy as jnp
from jax.experimental import pallas as pl

_EMB = 1024
_E = 16
_BLK = 2048


def _gate_kernel(h_ref, u_ref, W_ref, b_ref, o_ref):
    h = h_ref[...]
    u = u_ref[...]
    Wh = W_ref[:_EMB, :]
    Wu = W_ref[_EMB:, :]
    g = (
        jax.lax.dot(h, Wh, preferred_element_type=jnp.float32)
        + jax.lax.dot(u, Wu, preferred_element_type=jnp.float32)
        + b_ref[...]
    )
    m = jnp.max(g, axis=-1, keepdims=True)
    iota = jax.lax.broadcasted_iota(jnp.int32, g.shape, 1)
    i1 = jnp.min(jnp.where(g == m, iota, _E), axis=-1, keepdims=True)
    e = jnp.exp(g - m)
    e_rest = jnp.where(iota == i1, -1.0, e)
    e2 = jnp.max(e_rest, axis=-1, keepdims=True)
    S = jnp.sum(e, axis=-1, keepdims=True)
    r = 1.0 / (1.0 + e2 + 1e-9 * S)
    keep = (iota == i1) | (e_rest >= e2)
    o_ref[...] = jnp.where(keep, e * r, 0.0)


@jax.jit
def kernel(h, u, W, b):
    n = h.shape[0]
    return pl.pallas_call(
        _gate_kernel,
        grid=(n // _BLK,),
        in_specs=[
            pl.BlockSpec((_BLK, _EMB), lambda i: (i, 0)),
            pl.BlockSpec((_BLK, u.shape[1]), lambda i: (i, 0)),
            pl.BlockSpec(W.shape, lambda i: (0, 0)),
            pl.BlockSpec(b.shape, lambda i: (0,)),
        ],
        out_specs=pl.BlockSpec((_BLK, _E), lambda i: (i, 0)),
        out_shape=jax.ShapeDtypeStruct((n, _E), jnp.float32),
    )(h, u, W, b)
